# Initial kernel scaffold; baseline (speedup 1.0000x reference)
import jax
import jax.numpy as jnp
from jax.experimental import pallas as pl


def kernel(graph_node, edge_index, W0, b0, W1, b1):
    raise NotImplementedError("write your pallas kernel here")


# speedup vs baseline: 107.3319x; 107.3319x over previous
import dataclasses
import functools

import jax
import jax.numpy as jnp
from jax import lax
from jax.experimental import pallas as pl
from jax.experimental.pallas import tpu as pltpu
from jax.experimental.pallas import tpu_sc as plsc

N_NODES = 10000
N_EDGES = 320000
D = 128

NC = 2
NS = 16
NW = NC * NS
LANES = 16

N_PAD = 10240
DUMMY = N_NODES
SLICE = N_PAD // NS

CHUNK = 2048
ROWS = 16
E_PAD = 327680
CHUNKS_PER_TILE = E_PAD // (NW * CHUNK)
EDGES_PER_TILE = E_PAD // NW

_mesh = plsc.VectorSubcoreMesh(core_axis_name="c", subcore_axis_name="s")

_cp = pltpu.CompilerParams()
if "needs_layout_passes" in pltpu.CompilerParams.__dataclass_fields__:
    _cp = dataclasses.replace(_cp, needs_layout_passes=False)


def _zero_my_spmem_slice(zbuf, sp, sid):
    @pl.loop(0, SLICE // LANES)
    def _(m):
        zbuf[pl.ds(m * LANES, LANES)] = jnp.zeros((LANES,), jnp.float32)

    pltpu.sync_copy(zbuf, sp.at[pl.ds(sid * SLICE, SLICE)])


@functools.partial(
    pl.kernel,
    out_type=jax.ShapeDtypeStruct((NC, N_PAD), jnp.float32),
    mesh=_mesh,
    compiler_params=_cp,
    scratch_types=[
        pltpu.VMEM((CHUNK,), jnp.int32),
        pltpu.VMEM((CHUNK,), jnp.int32),
        pltpu.VMEM((ROWS, 128), jnp.int32),
        pltpu.VMEM((128,), jnp.float32),
        pltpu.VMEM((SLICE,), jnp.float32),
        pltpu.VMEM_SHARED((N_PAD,), jnp.float32),
    ],
)
def _sc_degree(src_hbm, dst_hbm, degp_hbm, src_b, dst_b, idx_b, ones_b, zbuf, deg_sp):
    cid = lax.axis_index("c")
    sid = lax.axis_index("s")
    wid = cid * NS + sid

    _zero_my_spmem_slice(zbuf, deg_sp, sid)

    @pl.loop(0, 128 // LANES)
    def _(m):
        ones_b[pl.ds(m * LANES, LANES)] = jnp.ones((LANES,), jnp.float32)

    plsc.subcore_barrier()

    base = wid * EDGES_PER_TILE

    @pl.loop(0, CHUNKS_PER_TILE)
    def _(k):
        pltpu.sync_copy(src_hbm.at[pl.ds(base + k * CHUNK, CHUNK)], src_b)
        pltpu.sync_copy(dst_hbm.at[pl.ds(base + k * CHUNK, CHUNK)], dst_b)

        @pl.loop(0, ROWS)
        def _(r):
            @pl.loop(0, 128 // LANES)
            def _(c):
                off = r * 128 + c * LANES
                s16 = src_b[pl.ds(off, LANES)]
                d16 = dst_b[pl.ds(off, LANES)]
                idx_b[r, pl.ds(c * LANES, LANES)] = jnp.where(s16 == d16, DUMMY, d16)

        for j in range(ROWS):
            pltpu.sync_copy(ones_b, deg_sp.at[idx_b.at[j]], add=True)

    plsc.subcore_barrier()
    pltpu.sync_copy(deg_sp.at[pl.ds(sid * SLICE, SLICE)],
                    degp_hbm.at[cid, pl.ds(sid * SLICE, SLICE)])


@functools.partial(
    pl.kernel,
    out_type=jax.ShapeDtypeStruct((NC, N_PAD), jnp.float32),
    mesh=_mesh,
    compiler_params=_cp,
    scratch_types=[
        pltpu.VMEM((CHUNK,), jnp.int32),
        pltpu.VMEM((CHUNK,), jnp.int32),
        pltpu.VMEM((ROWS, 128), jnp.int32),
        pltpu.VMEM((ROWS, 128), jnp.float32),
        pltpu.VMEM((N_PAD,), jnp.float32),
        pltpu.VMEM((SLICE,), jnp.float32),
        pltpu.VMEM_SHARED((N_PAD,), jnp.float32),
    ],
)
def _sc_tsum(src_hbm, dst_hbm, dinv_hbm, tp_hbm,
             src_b, dst_b, idx_b, val_b, dinv_b, zbuf, t_sp):
    cid = lax.axis_index("c")
    sid = lax.axis_index("s")
    wid = cid * NS + sid

    _zero_my_spmem_slice(zbuf, t_sp, sid)
    pltpu.sync_copy(dinv_hbm, dinv_b)
    plsc.subcore_barrier()

    base = wid * EDGES_PER_TILE

    @pl.loop(0, CHUNKS_PER_TILE)
    def _(k):
        pltpu.sync_copy(src_hbm.at[pl.ds(base + k * CHUNK, CHUNK)], src_b)
        pltpu.sync_copy(dst_hbm.at[pl.ds(base + k * CHUNK, CHUNK)], dst_b)

        @pl.loop(0, ROWS)
        def _(r):
            @pl.loop(0, 128 // LANES)
            def _(c):
                off = r * 128 + c * LANES
                s16 = src_b[pl.ds(off, LANES)]
                d16 = dst_b[pl.ds(off, LANES)]
                gv = plsc.load_gather(dinv_b, [d16])
                idx_b[r, pl.ds(c * LANES, LANES)] = s16
                val_b[r, pl.ds(c * LANES, LANES)] = jnp.where(s16 == d16, 0.0, gv)

        for j in range(ROWS):
            pltpu.sync_copy(val_b.at[j], t_sp.at[idx_b.at[j]], add=True)

    plsc.subcore_barrier()
    pltpu.sync_copy(t_sp.at[pl.ds(sid * SLICE, SLICE)],
                    tp_hbm.at[cid, pl.ds(sid * SLICE, SLICE)])


def _mm0_body(g_ref, w_ref, o_ref):
    o_ref[...] = lax.dot_general(g_ref[...], w_ref[...],
                                 (((1,), (1,)), ((), ())),
                                 preferred_element_type=jnp.float32)


def _dinv_body(degp_ref, o_ref):
    deg = degp_ref[0, :] + degp_ref[1, :] + 1.0
    o_ref[0, :] = lax.rsqrt(deg)


def _s_body(dinv_ref, tp_ref, o_ref):
    dv = dinv_ref[0, :]
    t = tp_ref[0, :] + tp_ref[1, :]
    o_ref[0, :] = dv * (t + dv)


def _dense_body(x1_ref, s_ref, w1_ref, b0_ref, b1_ref, o_ref):
    s = s_ref[...]
    x = x1_ref[...] * s + b0_ref[...]
    m = jnp.mean(x, axis=0, keepdims=True)
    xc = x - m
    v = jnp.mean(xc * xc, axis=0, keepdims=True)
    h = jnp.maximum(xc * lax.rsqrt(v + 1e-5), 0.0)
    y = lax.dot_general(h, w1_ref[...],
                        (((1,), (1,)), ((), ())),
                        preferred_element_type=jnp.float32)
    o_ref[...] = y * s + b1_ref[...]


def kernel(graph_node, edge_index, W0, b0, W1, b1):
    pad = jnp.zeros((E_PAD - N_EDGES,), jnp.int32)
    src = jnp.concatenate([edge_index[0], pad])
    dst = jnp.concatenate([edge_index[1], pad])

    degp = _sc_degree(src, dst)

    x1 = pl.pallas_call(
        _mm0_body,
        out_shape=jax.ShapeDtypeStruct((N_NODES, D), jnp.float32),
    )(graph_node, W0)

    dinv = pl.pallas_call(
        _dinv_body,
        out_shape=jax.ShapeDtypeStruct((1, N_PAD), jnp.float32),
    )(degp)

    tp = _sc_tsum(src, dst, dinv.reshape(N_PAD))

    s_row = pl.pallas_call(
        _s_body,
        out_shape=jax.ShapeDtypeStruct((1, N_PAD), jnp.float32),
    )(dinv, tp)

    s_col = s_row[0, :N_NODES][:, None]

    out = pl.pallas_call(
        _dense_body,
        out_shape=jax.ShapeDtypeStruct((N_NODES, D), jnp.float32),
    )(x1, s_col, W1, b0[None, :], b1[None, :])
    return out


# speedup vs baseline: 147.8101x; 1.3771x over previous
import dataclasses
import functools

import jax
import jax.numpy as jnp
from jax import lax
from jax.experimental import pallas as pl
from jax.experimental.pallas import tpu as pltpu
from jax.experimental.pallas import tpu_sc as plsc

N_NODES = 10000
N_EDGES = 320000
D = 128

NC = 2
NS = 16
NW = NC * NS
LANES = 16

N_PAD = 10240
EDGES_PER_TILE = N_EDGES // NW
GROUPS = EDGES_PER_TILE // LANES

_mesh = plsc.VectorSubcoreMesh(core_axis_name="c", subcore_axis_name="s")

_cp = pltpu.CompilerParams()
if "needs_layout_passes" in pltpu.CompilerParams.__dataclass_fields__:
    _cp = dataclasses.replace(_cp, needs_layout_passes=False)


@functools.partial(
    pl.kernel,
    out_type=jax.ShapeDtypeStruct((NW, N_PAD), jnp.float32),
    mesh=_mesh,
    compiler_params=_cp,
    scratch_types=[
        pltpu.VMEM((EDGES_PER_TILE,), jnp.int32),
        pltpu.VMEM((EDGES_PER_TILE,), jnp.int32),
        pltpu.VMEM((N_PAD,), jnp.float32),
    ],
)
def _sc_degree(ei_hbm, degp_hbm, src_b, dst_b, hist):
    cid = lax.axis_index("c")
    sid = lax.axis_index("s")
    wid = cid * NS + sid
    base = wid * EDGES_PER_TILE

    @pl.loop(0, N_PAD // LANES)
    def _(m):
        hist[pl.ds(m * LANES, LANES)] = jnp.zeros((LANES,), jnp.float32)

    pltpu.sync_copy(ei_hbm.at[pl.ds(base, EDGES_PER_TILE)], src_b)
    pltpu.sync_copy(ei_hbm.at[pl.ds(N_EDGES + base, EDGES_PER_TILE)], dst_b)

    ones = jnp.ones((LANES,), jnp.float32)

    @pl.loop(0, GROUPS)
    def _(g):
        s16 = src_b[pl.ds(g * LANES, LANES)]
        d16 = dst_b[pl.ds(g * LANES, LANES)]
        plsc.addupdate_scatter(hist, [d16], ones, mask=s16 != d16)

    pltpu.sync_copy(hist, degp_hbm.at[wid])


@functools.partial(
    pl.kernel,
    out_type=jax.ShapeDtypeStruct((NW, N_PAD), jnp.float32),
    mesh=_mesh,
    compiler_params=_cp,
    scratch_types=[
        pltpu.VMEM((EDGES_PER_TILE,), jnp.int32),
        pltpu.VMEM((EDGES_PER_TILE,), jnp.int32),
        pltpu.VMEM((N_PAD,), jnp.float32),
        pltpu.VMEM((N_PAD,), jnp.float32),
    ],
)
def _sc_tsum(ei_hbm, dinv_hbm, tp_hbm, src_b, dst_b, dinv_b, hist):
    cid = lax.axis_index("c")
    sid = lax.axis_index("s")
    wid = cid * NS + sid
    base = wid * EDGES_PER_TILE

    @pl.loop(0, N_PAD // LANES)
    def _(m):
        hist[pl.ds(m * LANES, LANES)] = jnp.zeros((LANES,), jnp.float32)

    pltpu.sync_copy(ei_hbm.at[pl.ds(base, EDGES_PER_TILE)], src_b)
    pltpu.sync_copy(ei_hbm.at[pl.ds(N_EDGES + base, EDGES_PER_TILE)], dst_b)
    pltpu.sync_copy(dinv_hbm, dinv_b)

    @pl.loop(0, GROUPS)
    def _(g):
        s16 = src_b[pl.ds(g * LANES, LANES)]
        d16 = dst_b[pl.ds(g * LANES, LANES)]
        gv = plsc.load_gather(dinv_b, [d16])
        plsc.addupdate_scatter(hist, [s16], gv, mask=s16 != d16)

    pltpu.sync_copy(hist, tp_hbm.at[wid])


def _mm0_body(g_ref, w_ref, o_ref):
    o_ref[...] = lax.dot_general(g_ref[...], w_ref[...],
                                 (((1,), (1,)), ((), ())),
                                 preferred_element_type=jnp.float32)


def _dinv_body(degp_ref, o_ref):
    deg = jnp.sum(degp_ref[...], axis=0) + 1.0
    o_ref[0, :] = lax.rsqrt(deg)


def _s_body(dinv_ref, tp_ref, o_ref):
    dv = dinv_ref[0, :]
    t = jnp.sum(tp_ref[...], axis=0)
    o_ref[0, :] = dv * (t + dv)


def _dense_body(x1_ref, s_ref, w1_ref, b0_ref, b1_ref, o_ref):
    s = s_ref[...]
    x = x1_ref[...] * s + b0_ref[...]
    m = jnp.mean(x, axis=0, keepdims=True)
    xc = x - m
    v = jnp.mean(xc * xc, axis=0, keepdims=True)
    h = jnp.maximum(xc * lax.rsqrt(v + 1e-5), 0.0)
    y = lax.dot_general(h, w1_ref[...],
                        (((1,), (1,)), ((), ())),
                        preferred_element_type=jnp.float32)
    o_ref[...] = y * s + b1_ref[...]


def kernel(graph_node, edge_index, W0, b0, W1, b1):
    ei_flat = edge_index.reshape(-1)
    degp = _sc_degree(ei_flat)

    x1 = pl.pallas_call(
        _mm0_body,
        out_shape=jax.ShapeDtypeStruct((N_NODES, D), jnp.float32),
    )(graph_node, W0)

    dinv = pl.pallas_call(
        _dinv_body,
        out_shape=jax.ShapeDtypeStruct((1, N_PAD), jnp.float32),
    )(degp)

    tp = _sc_tsum(ei_flat, dinv.reshape(N_PAD))

    s_row = pl.pallas_call(
        _s_body,
        out_shape=jax.ShapeDtypeStruct((1, N_PAD), jnp.float32),
    )(dinv, tp)

    s_col = s_row[0, :N_NODES][:, None]

    out = pl.pallas_call(
        _dense_body,
        out_shape=jax.ShapeDtypeStruct((N_NODES, D), jnp.float32),
    )(x1, s_col, W1, b0[None, :], b1[None, :])
    return out


# speedup vs baseline: 155.2665x; 1.0504x over previous
import dataclasses
import functools

import jax
import jax.numpy as jnp
from jax import lax
from jax.experimental import pallas as pl
from jax.experimental.pallas import tpu as pltpu
from jax.experimental.pallas import tpu_sc as plsc

N_NODES = 10000
N_EDGES = 320000
D = 128

NC = 2
NS = 16
NW = NC * NS
LANES = 16

N_PAD = 10240
EDGES_MAIN = 9984
EDGES_WIN = 10496
UNROLL = 4
GROUPS = EDGES_MAIN // LANES
OUTER = GROUPS // UNROLL
TAIL_GROUPS = (EDGES_WIN - EDGES_MAIN) // LANES

_mesh = plsc.VectorSubcoreMesh(core_axis_name="c", subcore_axis_name="s")

_cp = pltpu.CompilerParams()
if "needs_layout_passes" in pltpu.CompilerParams.__dataclass_fields__:
    _cp = dataclasses.replace(_cp, needs_layout_passes=False)


@functools.partial(
    pl.kernel,
    out_type=jax.ShapeDtypeStruct((NW, N_PAD), jnp.float32),
    mesh=_mesh,
    compiler_params=_cp,
    scratch_types=[
        pltpu.VMEM((2, EDGES_WIN), jnp.int32),
        pltpu.VMEM((N_PAD,), jnp.float32),
    ],
)
def _sc_degree(ei_hbm, degp_hbm, edge_b, hist):
    cid = lax.axis_index("c")
    sid = lax.axis_index("s")
    wid = cid * NS + sid
    base = wid * EDGES_MAIN

    @pl.loop(0, N_PAD // LANES)
    def _(m):
        hist[pl.ds(m * LANES, LANES)] = jnp.zeros((LANES,), jnp.float32)

    pltpu.sync_copy(ei_hbm.at[:, pl.ds(base, EDGES_WIN)], edge_b)

    ones = jnp.ones((LANES,), jnp.float32)

    def body(off):
        s16 = edge_b[0, pl.ds(off, LANES)]
        d16 = edge_b[1, pl.ds(off, LANES)]
        plsc.addupdate_scatter(hist, [d16], ones, mask=s16 != d16)

    @pl.loop(0, OUTER)
    def _(g):
        for u in range(UNROLL):
            body(g * (UNROLL * LANES) + u * LANES)

    @pl.when(wid == NW - 1)
    def _():
        @pl.loop(0, TAIL_GROUPS)
        def _(g):
            body(EDGES_MAIN + g * LANES)

    pltpu.sync_copy(hist, degp_hbm.at[wid])


@functools.partial(
    pl.kernel,
    out_type=jax.ShapeDtypeStruct((NW, N_PAD), jnp.float32),
    mesh=_mesh,
    compiler_params=_cp,
    scratch_types=[
        pltpu.VMEM((2, EDGES_WIN), jnp.int32),
        pltpu.VMEM((N_PAD,), jnp.float32),
        pltpu.VMEM((N_PAD,), jnp.float32),
    ],
)
def _sc_tsum(ei_hbm, dinv_hbm, tp_hbm, edge_b, dinv_b, hist):
    cid = lax.axis_index("c")
    sid = lax.axis_index("s")
    wid = cid * NS + sid
    base = wid * EDGES_MAIN

    @pl.loop(0, N_PAD // LANES)
    def _(m):
        hist[pl.ds(m * LANES, LANES)] = jnp.zeros((LANES,), jnp.float32)

    pltpu.sync_copy(ei_hbm.at[:, pl.ds(base, EDGES_WIN)], edge_b)
    pltpu.sync_copy(dinv_hbm, dinv_b)

    def body(off):
        s16 = edge_b[0, pl.ds(off, LANES)]
        d16 = edge_b[1, pl.ds(off, LANES)]
        gv = plsc.load_gather(dinv_b, [d16])
        plsc.addupdate_scatter(hist, [s16], gv, mask=s16 != d16)

    @pl.loop(0, OUTER)
    def _(g):
        for u in range(UNROLL):
            body(g * (UNROLL * LANES) + u * LANES)

    @pl.when(wid == NW - 1)
    def _():
        @pl.loop(0, TAIL_GROUPS)
        def _(g):
            body(EDGES_MAIN + g * LANES)

    pltpu.sync_copy(hist, tp_hbm.at[wid])


def _mm0_body(g_ref, w_ref, o_ref):
    o_ref[...] = lax.dot_general(g_ref[...], w_ref[...],
                                 (((1,), (1,)), ((), ())),
                                 preferred_element_type=jnp.float32)


def _dinv_body(degp_ref, o_ref):
    deg = jnp.sum(degp_ref[...], axis=0) + 1.0
    o_ref[0, :] = lax.rsqrt(deg)


def _s_body(dinv_ref, tp_ref, o_ref):
    dv = dinv_ref[0, :]
    t = jnp.sum(tp_ref[...], axis=0)
    o_ref[0, :] = dv * (t + dv)


def _dense_body(x1_ref, s_ref, w1_ref, b0_ref, b1_ref, o_ref):
    s = s_ref[...]
    x = x1_ref[...] * s + b0_ref[...]
    m = jnp.mean(x, axis=0, keepdims=True)
    xc = x - m
    v = jnp.mean(xc * xc, axis=0, keepdims=True)
    h = jnp.maximum(xc * lax.rsqrt(v + 1e-5), 0.0)
    y = lax.dot_general(h, w1_ref[...],
                        (((1,), (1,)), ((), ())),
                        preferred_element_type=jnp.float32)
    o_ref[...] = y * s + b1_ref[...]


def kernel(graph_node, edge_index, W0, b0, W1, b1):
    degp = _sc_degree(edge_index)

    x1 = pl.pallas_call(
        _mm0_body,
        out_shape=jax.ShapeDtypeStruct((N_NODES, D), jnp.float32),
    )(graph_node, W0)

    dinv = pl.pallas_call(
        _dinv_body,
        out_shape=jax.ShapeDtypeStruct((1, N_PAD), jnp.float32),
    )(degp)

    tp = _sc_tsum(edge_index, dinv.reshape(N_PAD))

    s_row = pl.pallas_call(
        _s_body,
        out_shape=jax.ShapeDtypeStruct((1, N_PAD), jnp.float32),
    )(dinv, tp)

    s_col = s_row[0, :N_NODES][:, None]

    out = pl.pallas_call(
        _dense_body,
        out_shape=jax.ShapeDtypeStruct((N_NODES, D), jnp.float32),
    )(x1, s_col, W1, b0[None, :], b1[None, :])
    return out


# speedup vs baseline: 166.1911x; 1.0704x over previous
import dataclasses
import functools

import jax
import jax.numpy as jnp
from jax import lax
from jax.experimental import pallas as pl
from jax.experimental.pallas import tpu as pltpu
from jax.experimental.pallas import tpu_sc as plsc

N_NODES = 10000
N_EDGES = 320000
D = 128

NC = 2
NS = 16
NW = NC * NS
LANES = 16

N_PAD = 10240
EDGES_MAIN = 9984
EDGES_WIN = 10496
UNROLL = 4
GROUPS = EDGES_MAIN // LANES
OUTER = GROUPS // UNROLL
TAIL_GROUPS = (EDGES_WIN - EDGES_MAIN) // LANES

_mesh = plsc.VectorSubcoreMesh(core_axis_name="c", subcore_axis_name="s")

_cp = pltpu.CompilerParams()
if "needs_layout_passes" in pltpu.CompilerParams.__dataclass_fields__:
    _cp = dataclasses.replace(_cp, needs_layout_passes=False)


@functools.partial(
    pl.kernel,
    out_type=jax.ShapeDtypeStruct((2 * NW, N_PAD), jnp.float32),
    mesh=_mesh,
    compiler_params=_cp,
    scratch_types=[
        pltpu.VMEM((2, EDGES_WIN), jnp.int32),
        pltpu.VMEM((N_PAD,), jnp.float32),
        pltpu.VMEM((N_PAD,), jnp.float32),
    ],
)
def _sc_degree(ei_hbm, degp_hbm, edge_b, hist_a, hist_b):
    cid = lax.axis_index("c")
    sid = lax.axis_index("s")
    wid = cid * NS + sid
    base = wid * EDGES_MAIN

    @pl.loop(0, N_PAD // LANES)
    def _(m):
        z = jnp.zeros((LANES,), jnp.float32)
        hist_a[pl.ds(m * LANES, LANES)] = z
        hist_b[pl.ds(m * LANES, LANES)] = z

    pltpu.sync_copy(ei_hbm.at[:, pl.ds(base, EDGES_WIN)], edge_b)

    ones = jnp.ones((LANES,), jnp.float32)

    def body(off, hist):
        s16 = edge_b[0, pl.ds(off, LANES)]
        d16 = edge_b[1, pl.ds(off, LANES)]
        plsc.addupdate_scatter(hist, [d16], ones, mask=s16 != d16)

    @pl.loop(0, OUTER)
    def _(g):
        for u in range(UNROLL):
            body(g * (UNROLL * LANES) + u * LANES, hist_a if u % 2 == 0 else hist_b)

    @pl.when(wid == NW - 1)
    def _():
        @pl.loop(0, TAIL_GROUPS)
        def _(g):
            body(EDGES_MAIN + g * LANES, hist_a)

    pltpu.sync_copy(hist_a, degp_hbm.at[wid])
    pltpu.sync_copy(hist_b, degp_hbm.at[NW + wid])


@functools.partial(
    pl.kernel,
    out_type=jax.ShapeDtypeStruct((2 * NW, N_PAD), jnp.float32),
    mesh=_mesh,
    compiler_params=_cp,
    scratch_types=[
        pltpu.VMEM((2, EDGES_WIN), jnp.int32),
        pltpu.VMEM((N_PAD,), jnp.float32),
        pltpu.VMEM((N_PAD,), jnp.float32),
        pltpu.VMEM((N_PAD,), jnp.float32),
    ],
)
def _sc_tsum(ei_hbm, dinv_hbm, tp_hbm, edge_b, dinv_b, hist_a, hist_b):
    cid = lax.axis_index("c")
    sid = lax.axis_index("s")
    wid = cid * NS + sid
    base = wid * EDGES_MAIN

    @pl.loop(0, N_PAD // LANES)
    def _(m):
        z = jnp.zeros((LANES,), jnp.float32)
        hist_a[pl.ds(m * LANES, LANES)] = z
        hist_b[pl.ds(m * LANES, LANES)] = z

    pltpu.sync_copy(ei_hbm.at[:, pl.ds(base, EDGES_WIN)], edge_b)
    pltpu.sync_copy(dinv_hbm, dinv_b)

    def body(off, hist):
        s16 = edge_b[0, pl.ds(off, LANES)]
        d16 = edge_b[1, pl.ds(off, LANES)]
        gv = plsc.load_gather(dinv_b, [d16])
        plsc.addupdate_scatter(hist, [s16], gv, mask=s16 != d16)

    @pl.loop(0, OUTER)
    def _(g):
        for u in range(UNROLL):
            body(g * (UNROLL * LANES) + u * LANES, hist_a if u % 2 == 0 else hist_b)

    @pl.when(wid == NW - 1)
    def _():
        @pl.loop(0, TAIL_GROUPS)
        def _(g):
            body(EDGES_MAIN + g * LANES, hist_a)

    pltpu.sync_copy(hist_a, tp_hbm.at[wid])
    pltpu.sync_copy(hist_b, tp_hbm.at[NW + wid])


def _mm0_body(g_ref, w_ref, o_ref):
    o_ref[...] = lax.dot_general(g_ref[...], w_ref[...],
                                 (((1,), (1,)), ((), ())),
                                 preferred_element_type=jnp.float32)


def _dinv_body(degp_ref, o_ref):
    deg = jnp.sum(degp_ref[...], axis=0) + 1.0
    o_ref[0, :] = lax.rsqrt(deg)


def _dense_body(x1_ref, dinv_ref, tp_ref, w1_ref, b0_ref, b1_ref, o_ref):
    dv = dinv_ref[...]
    t = jnp.sum(tp_ref[...], axis=0, keepdims=True)
    s_row = dv * (t + dv)
    s = lax.transpose(s_row, (1, 0))[:N_NODES, :]
    x = x1_ref[...] * s + b0_ref[...]
    m = jnp.mean(x, axis=0, keepdims=True)
    xc = x - m
    v = jnp.mean(xc * xc, axis=0, keepdims=True)
    h = jnp.maximum(xc * lax.rsqrt(v + 1e-5), 0.0)
    y = lax.dot_general(h, w1_ref[...],
                        (((1,), (1,)), ((), ())),
                        preferred_element_type=jnp.float32)
    o_ref[...] = y * s + b1_ref[...]


def kernel(graph_node, edge_index, W0, b0, W1, b1):
    degp = _sc_degree(edge_index)

    x1 = pl.pallas_call(
        _mm0_body,
        out_shape=jax.ShapeDtypeStruct((N_NODES, D), jnp.float32),
    )(graph_node, W0)

    dinv = pl.pallas_call(
        _dinv_body,
        out_shape=jax.ShapeDtypeStruct((1, N_PAD), jnp.float32),
    )(degp)

    tp = _sc_tsum(edge_index, dinv.reshape(N_PAD))

    out = pl.pallas_call(
        _dense_body,
        out_shape=jax.ShapeDtypeStruct((N_NODES, D), jnp.float32),
    )(x1, dinv, tp, W1, b0[None, :], b1[None, :])
    return out


# speedup vs baseline: 192.7222x; 1.1596x over previous
import dataclasses
import functools

import jax
import jax.numpy as jnp
from jax import lax
from jax.experimental import pallas as pl
from jax.experimental.pallas import tpu as pltpu
from jax.experimental.pallas import tpu_sc as plsc

N_NODES = 10000
N_EDGES = 320000
D = 128

NC = 2
NS = 16
NW = NC * NS
LANES = 16

N_PAD = 10240
EDGES_MAIN = 9984
EDGES_WIN = 10496
UNROLL = 4
GROUPS = EDGES_MAIN // LANES
OUTER = GROUPS // UNROLL
TAIL_GROUPS = (EDGES_WIN - EDGES_MAIN) // LANES

_mesh = plsc.VectorSubcoreMesh(core_axis_name="c", subcore_axis_name="s")

_cp = pltpu.CompilerParams()
if "needs_layout_passes" in pltpu.CompilerParams.__dataclass_fields__:
    _cp = dataclasses.replace(_cp, needs_layout_passes=False)


@functools.partial(
    pl.kernel,
    out_type=jax.ShapeDtypeStruct((2 * NW, N_PAD), jnp.float32),
    mesh=_mesh,
    compiler_params=_cp,
    scratch_types=[
        pltpu.VMEM((2, EDGES_WIN), jnp.int32),
        pltpu.VMEM((N_PAD,), jnp.float32),
        pltpu.VMEM((N_PAD,), jnp.float32),
    ],
)
def _sc_degree(ei_hbm, degp_hbm, edge_b, hist_a, hist_b):
    cid = lax.axis_index("c")
    sid = lax.axis_index("s")
    wid = cid * NS + sid
    base = wid * EDGES_MAIN

    @pl.loop(0, N_PAD // LANES)
    def _(m):
        z = jnp.zeros((LANES,), jnp.float32)
        hist_a[pl.ds(m * LANES, LANES)] = z
        hist_b[pl.ds(m * LANES, LANES)] = z

    pltpu.sync_copy(ei_hbm.at[:, pl.ds(base, EDGES_WIN)], edge_b)

    ones = jnp.ones((LANES,), jnp.float32)

    def body(off, hist):
        s16 = edge_b[0, pl.ds(off, LANES)]
        d16 = edge_b[1, pl.ds(off, LANES)]
        plsc.addupdate_scatter(hist, [d16], ones, mask=s16 != d16)

    @plsc.parallel_loop(0, GROUPS // 2, unroll=UNROLL)
    def _(g):
        off = g * (2 * LANES)
        body(off, hist_a)
        body(off + LANES, hist_b)

    @pl.when(wid == NW - 1)
    def _():
        @pl.loop(0, TAIL_GROUPS)
        def _(g):
            body(EDGES_MAIN + g * LANES, hist_a)

    pltpu.sync_copy(hist_a, degp_hbm.at[wid])
    pltpu.sync_copy(hist_b, degp_hbm.at[NW + wid])


@functools.partial(
    pl.kernel,
    out_type=jax.ShapeDtypeStruct((2 * NW, N_PAD), jnp.float32),
    mesh=_mesh,
    compiler_params=_cp,
    scratch_types=[
        pltpu.VMEM((2, EDGES_WIN), jnp.int32),
        pltpu.VMEM((N_PAD,), jnp.float32),
        pltpu.VMEM((N_PAD,), jnp.float32),
        pltpu.VMEM((N_PAD,), jnp.float32),
    ],
)
def _sc_tsum(ei_hbm, dinv_hbm, tp_hbm, edge_b, dinv_b, hist_a, hist_b):
    cid = lax.axis_index("c")
    sid = lax.axis_index("s")
    wid = cid * NS + sid
    base = wid * EDGES_MAIN

    @pl.loop(0, N_PAD // LANES)
    def _(m):
        z = jnp.zeros((LANES,), jnp.float32)
        hist_a[pl.ds(m * LANES, LANES)] = z
        hist_b[pl.ds(m * LANES, LANES)] = z

    pltpu.sync_copy(ei_hbm.at[:, pl.ds(base, EDGES_WIN)], edge_b)
    pltpu.sync_copy(dinv_hbm, dinv_b)

    def body(off, hist):
        s16 = edge_b[0, pl.ds(off, LANES)]
        d16 = edge_b[1, pl.ds(off, LANES)]
        gv = plsc.load_gather(dinv_b, [d16])
        plsc.addupdate_scatter(hist, [s16], gv, mask=s16 != d16)

    @plsc.parallel_loop(0, GROUPS // 2, unroll=UNROLL)
    def _(g):
        off = g * (2 * LANES)
        body(off, hist_a)
        body(off + LANES, hist_b)

    @pl.when(wid == NW - 1)
    def _():
        @pl.loop(0, TAIL_GROUPS)
        def _(g):
            body(EDGES_MAIN + g * LANES, hist_a)

    pltpu.sync_copy(hist_a, tp_hbm.at[wid])
    pltpu.sync_copy(hist_b, tp_hbm.at[NW + wid])


def _mm0_body(g_ref, w_ref, o_ref):
    o_ref[...] = lax.dot_general(g_ref[...], w_ref[...],
                                 (((1,), (1,)), ((), ())),
                                 preferred_element_type=jnp.float32)


def _dinv_body(degp_ref, o_ref):
    deg = jnp.sum(degp_ref[...], axis=0) + 1.0
    o_ref[0, :] = lax.rsqrt(deg)


def _dense_body(x1_ref, dinv_ref, tp_ref, w1_ref, b0_ref, b1_ref, o_ref):
    dv = dinv_ref[...]
    t = jnp.sum(tp_ref[...], axis=0, keepdims=True)
    s_row = dv * (t + dv)
    s = lax.transpose(s_row, (1, 0))[:N_NODES, :]
    x = x1_ref[...] * s + b0_ref[...]
    m = jnp.mean(x, axis=0, keepdims=True)
    xc = x - m
    v = jnp.mean(xc * xc, axis=0, keepdims=True)
    h = jnp.maximum(xc * lax.rsqrt(v + 1e-5), 0.0)
    y = lax.dot_general(h, w1_ref[...],
                        (((1,), (1,)), ((), ())),
                        preferred_element_type=jnp.float32)
    o_ref[...] = y * s + b1_ref[...]


def kernel(graph_node, edge_index, W0, b0, W1, b1):
    degp = _sc_degree(edge_index)

    x1 = pl.pallas_call(
        _mm0_body,
        out_shape=jax.ShapeDtypeStruct((N_NODES, D), jnp.float32),
    )(graph_node, W0)

    dinv = pl.pallas_call(
        _dinv_body,
        out_shape=jax.ShapeDtypeStruct((1, N_PAD), jnp.float32),
    )(degp)

    tp = _sc_tsum(edge_index, dinv.reshape(N_PAD))

    out = pl.pallas_call(
        _dense_body,
        out_shape=jax.ShapeDtypeStruct((N_NODES, D), jnp.float32),
    )(x1, dinv, tp, W1, b0[None, :], b1[None, :])
    return out


# speedup vs baseline: 204.5383x; 1.0613x over previous
import dataclasses
import functools

import jax
import jax.numpy as jnp
from jax import lax
from jax.experimental import pallas as pl
from jax.experimental.pallas import tpu as pltpu
from jax.experimental.pallas import tpu_sc as plsc

N_NODES = 10000
N_EDGES = 320000
D = 128

NC = 2
NS = 16
NW = NC * NS
LANES = 16

N_PAD = 10240
EDGES_MAIN = 9984
EDGES_WIN = 10496
UNROLL = 8
GROUPS = EDGES_MAIN // LANES
OUTER = GROUPS // UNROLL
TAIL_GROUPS = (EDGES_WIN - EDGES_MAIN) // LANES

_mesh = plsc.VectorSubcoreMesh(core_axis_name="c", subcore_axis_name="s")

_cp = pltpu.CompilerParams()
if "needs_layout_passes" in pltpu.CompilerParams.__dataclass_fields__:
    _cp = dataclasses.replace(_cp, needs_layout_passes=False)


@functools.partial(
    pl.kernel,
    out_type=jax.ShapeDtypeStruct((2 * NW, N_PAD), jnp.float32),
    mesh=_mesh,
    compiler_params=_cp,
    scratch_types=[
        pltpu.VMEM((2, EDGES_WIN), jnp.int32),
        pltpu.VMEM((N_PAD,), jnp.float32),
        pltpu.VMEM((N_PAD,), jnp.float32),
    ],
)
def _sc_degree(ei_hbm, degp_hbm, edge_b, hist_a, hist_b):
    cid = lax.axis_index("c")
    sid = lax.axis_index("s")
    wid = cid * NS + sid
    base = wid * EDGES_MAIN

    @plsc.parallel_loop(0, N_PAD // LANES, unroll=4)
    def _(m):
        z = jnp.zeros((LANES,), jnp.float32)
        hist_a[pl.ds(m * LANES, LANES)] = z
        hist_b[pl.ds(m * LANES, LANES)] = z

    pltpu.sync_copy(ei_hbm.at[:, pl.ds(base, EDGES_WIN)], edge_b)

    ones = jnp.ones((LANES,), jnp.float32)

    def body(off, hist):
        s16 = edge_b[0, pl.ds(off, LANES)]
        d16 = edge_b[1, pl.ds(off, LANES)]
        plsc.addupdate_scatter(hist, [d16], ones, mask=s16 != d16)

    @plsc.parallel_loop(0, GROUPS // 2, unroll=UNROLL)
    def _(g):
        off = g * (2 * LANES)
        body(off, hist_a)
        body(off + LANES, hist_b)

    @pl.when(wid == NW - 1)
    def _():
        @pl.loop(0, TAIL_GROUPS)
        def _(g):
            body(EDGES_MAIN + g * LANES, hist_a)

    pltpu.sync_copy(hist_a, degp_hbm.at[wid])
    pltpu.sync_copy(hist_b, degp_hbm.at[NW + wid])


@functools.partial(
    pl.kernel,
    out_type=jax.ShapeDtypeStruct((2 * NW, N_PAD), jnp.float32),
    mesh=_mesh,
    compiler_params=_cp,
    scratch_types=[
        pltpu.VMEM((2, EDGES_WIN), jnp.int32),
        pltpu.VMEM((N_PAD,), jnp.float32),
        pltpu.VMEM((N_PAD,), jnp.float32),
        pltpu.VMEM((N_PAD,), jnp.float32),
    ],
)
def _sc_tsum(ei_hbm, dinv_hbm, tp_hbm, edge_b, dinv_b, hist_a, hist_b):
    cid = lax.axis_index("c")
    sid = lax.axis_index("s")
    wid = cid * NS + sid
    base = wid * EDGES_MAIN

    @plsc.parallel_loop(0, N_PAD // LANES, unroll=4)
    def _(m):
        z = jnp.zeros((LANES,), jnp.float32)
        hist_a[pl.ds(m * LANES, LANES)] = z
        hist_b[pl.ds(m * LANES, LANES)] = z

    pltpu.sync_copy(ei_hbm.at[:, pl.ds(base, EDGES_WIN)], edge_b)
    pltpu.sync_copy(dinv_hbm, dinv_b)

    def body(off, hist):
        s16 = edge_b[0, pl.ds(off, LANES)]
        d16 = edge_b[1, pl.ds(off, LANES)]
        gv = plsc.load_gather(dinv_b, [d16])
        plsc.addupdate_scatter(hist, [s16], gv, mask=s16 != d16)

    @plsc.parallel_loop(0, GROUPS // 2, unroll=UNROLL)
    def _(g):
        off = g * (2 * LANES)
        body(off, hist_a)
        body(off + LANES, hist_b)

    @pl.when(wid == NW - 1)
    def _():
        @pl.loop(0, TAIL_GROUPS)
        def _(g):
            body(EDGES_MAIN + g * LANES, hist_a)

    pltpu.sync_copy(hist_a, tp_hbm.at[wid])
    pltpu.sync_copy(hist_b, tp_hbm.at[NW + wid])


def _mm0_body(g_ref, w_ref, o_ref):
    o_ref[...] = lax.dot_general(g_ref[...], w_ref[...],
                                 (((1,), (1,)), ((), ())),
                                 preferred_element_type=jnp.float32)


def _dinv_body(degp_ref, o_ref):
    deg = jnp.sum(degp_ref[...], axis=0) + 1.0
    o_ref[0, :] = lax.rsqrt(deg)


def _dense_body(x1_ref, dinv_ref, tp_ref, w1_ref, b0_ref, b1_ref, o_ref):
    dv = dinv_ref[...]
    t = jnp.sum(tp_ref[...], axis=0, keepdims=True)
    s_row = dv * (t + dv)
    s = lax.transpose(s_row, (1, 0))[:N_NODES, :]
    x = x1_ref[...] * s + b0_ref[...]
    m = jnp.mean(x, axis=0, keepdims=True)
    xc = x - m
    v = jnp.mean(xc * xc, axis=0, keepdims=True)
    h = jnp.maximum(xc * lax.rsqrt(v + 1e-5), 0.0)
    y = lax.dot_general(h, w1_ref[...],
                        (((1,), (1,)), ((), ())),
                        preferred_element_type=jnp.float32)
    o_ref[...] = y * s + b1_ref[...]


def kernel(graph_node, edge_index, W0, b0, W1, b1):
    degp = _sc_degree(edge_index)

    x1 = pl.pallas_call(
        _mm0_body,
        out_shape=jax.ShapeDtypeStruct((N_NODES, D), jnp.float32),
    )(graph_node, W0)

    dinv = pl.pallas_call(
        _dinv_body,
        out_shape=jax.ShapeDtypeStruct((1, N_PAD), jnp.float32),
    )(degp)

    tp = _sc_tsum(edge_index, dinv.reshape(N_PAD))

    out = pl.pallas_call(
        _dense_body,
        out_shape=jax.ShapeDtypeStruct((N_NODES, D), jnp.float32),
    )(x1, dinv, tp, W1, b0[None, :], b1[None, :])
    return out
